# Initial kernel scaffold; baseline (speedup 1.0000x reference)
import jax
import jax.numpy as jnp
from jax.experimental import pallas as pl


def kernel(node_attributes, edge_attributes, global_attributes, edge_index, W, b):
    raise NotImplementedError("write your pallas kernel here")


# speedup vs baseline: 9.8977x; 9.8977x over previous
import functools

import jax
import jax.numpy as jnp
from jax import lax
from jax.experimental import pallas as pl
from jax.experimental.pallas import tpu as pltpu
from jax.experimental.pallas import tpu_sc as plsc

N_NODES = 100000
N_EDGES = 3200000
D_NODE = 128
D_EDGE = 16
D_GLOBAL = 32

NC = 2
NS = 16
EDGES_PER_TILE = N_EDGES // NS
CHUNK = 800
N_CHUNKS = EDGES_PER_TILE // CHUNK
N_PAD = 100096
ROWS_PER_TILE = N_PAD // NS
ZROWS = 136
NZ = ROWS_PER_TILE // ZROWS


def _segment_sums(edge_attributes, src_idx, dst_idx):
    mesh = plsc.VectorSubcoreMesh(core_axis_name="c", subcore_axis_name="s")

    @functools.partial(
        pl.kernel,
        mesh=mesh,
        out_type=jax.ShapeDtypeStruct((2, N_PAD, D_EDGE), jnp.float32),
        scratch_types=[
            pltpu.VMEM_SHARED((N_PAD, D_EDGE), jnp.float32),
            pltpu.VMEM((CHUNK, D_EDGE), jnp.float32),
            pltpu.VMEM((CHUNK, D_EDGE), jnp.float32),
            pltpu.VMEM((CHUNK,), jnp.int32),
            pltpu.VMEM((CHUNK,), jnp.int32),
            pltpu.VMEM((ZROWS, D_EDGE), jnp.float32),
            pltpu.SemaphoreType.DMA,
            pltpu.SemaphoreType.DMA,
        ],
        compiler_params=pltpu.CompilerParams(use_tc_tiling_on_sc=False),
    )
    def seg(edge_hbm, src_hbm, dst_hbm, out_hbm, acc, ebuf0, ebuf1, ibuf0,
            ibuf1, zbuf, sem0, sem1):
        c = lax.axis_index("c")
        s = lax.axis_index("s")

        def zrow(i, carry):
            zbuf[i, :] = jnp.zeros((16,), jnp.float32)
            return carry

        lax.fori_loop(0, ZROWS, zrow, 0)
        row0 = s * ROWS_PER_TILE

        def zcopy(j, carry):
            pltpu.sync_copy(zbuf, acc.at[pl.ds(row0 + j * ZROWS, ZROWS), :])
            return carry

        lax.fori_loop(0, NZ, zcopy, 0)
        plsc.subcore_barrier()

        e_base = s * EDGES_PER_TILE
        ebufs = (ebuf0, ebuf1)
        ibufs = (ibuf0, ibuf1)
        sems = (sem0, sem1)

        def run_direction(idx_hbm):
            def start_load(k, slot):
                e0 = e_base + k * CHUNK
                pltpu.async_copy(edge_hbm.at[pl.ds(e0, CHUNK), :], ebufs[slot],
                                 sems[slot])
                pltpu.async_copy(idx_hbm.at[pl.ds(e0, CHUNK)], ibufs[slot],
                                 sems[slot])

            def wait_load(slot):
                pltpu.make_async_copy(edge_hbm.at[pl.ds(0, CHUNK), :],
                                      ebufs[slot], sems[slot]).wait()
                pltpu.make_async_copy(idx_hbm.at[pl.ds(0, CHUNK)],
                                      ibufs[slot], sems[slot]).wait()

            start_load(0, 0)

            def pair_body(g, carry):
                base = g * 2
                for slot in (0, 1):
                    k = base + slot
                    wait_load(slot)
                    pltpu.sync_copy(ebufs[slot], acc.at[ibufs[slot]], add=True)
                    nxt = k + 2

                    @pl.when(nxt < N_CHUNKS)
                    def _():
                        start_load(nxt, slot)

                return carry

            start_load(1, 1)
            lax.fori_loop(0, N_CHUNKS // 2, pair_body, 0)

        pl.when(c == 0)(lambda: run_direction(dst_hbm))
        pl.when(c == 1)(lambda: run_direction(src_hbm))
        plsc.subcore_barrier()

        pltpu.sync_copy(
            acc.at[pl.ds(row0, ROWS_PER_TILE), :],
            out_hbm.at[c, pl.ds(row0, ROWS_PER_TILE), :],
        )

    return seg(edge_attributes, src_idx, dst_idx)


def _node_update(seg, node_attributes, g_row, W_rec, W_sen, W_node, W_g, b_row):
    B = 4000
    grid = (N_NODES // B,)

    def mm(rec_ref, sen_ref, node_ref, g_ref, wr_ref, ws_ref, wn_ref, wg_ref,
           b_ref, out_ref):
        const = (
            jnp.dot(g_ref[...], wg_ref[...], preferred_element_type=jnp.float32)
            + b_ref[...]
        )
        out_ref[...] = (
            jnp.dot(node_ref[...], wn_ref[...], preferred_element_type=jnp.float32)
            + jnp.dot(rec_ref[0], wr_ref[...], preferred_element_type=jnp.float32)
            + jnp.dot(sen_ref[0], ws_ref[...], preferred_element_type=jnp.float32)
            + const
        )

    return pl.pallas_call(
        mm,
        grid=grid,
        in_specs=[
            pl.BlockSpec((1, B, D_EDGE), lambda i: (0, i, 0)),
            pl.BlockSpec((1, B, D_EDGE), lambda i: (1, i, 0)),
            pl.BlockSpec((B, D_NODE), lambda i: (i, 0)),
            pl.BlockSpec((1, D_GLOBAL), lambda i: (0, 0)),
            pl.BlockSpec((D_EDGE, D_NODE), lambda i: (0, 0)),
            pl.BlockSpec((D_EDGE, D_NODE), lambda i: (0, 0)),
            pl.BlockSpec((D_NODE, D_NODE), lambda i: (0, 0)),
            pl.BlockSpec((D_GLOBAL, D_NODE), lambda i: (0, 0)),
            pl.BlockSpec((1, D_NODE), lambda i: (0, 0)),
        ],
        out_specs=pl.BlockSpec((B, D_NODE), lambda i: (i, 0)),
        out_shape=jax.ShapeDtypeStruct((N_NODES, D_NODE), jnp.float32),
    )(seg, seg, node_attributes, g_row, W_rec, W_sen, W_node, W_g, b_row)


def kernel(node_attributes, edge_attributes, global_attributes, edge_index, W, b):
    src_idx = edge_index[0]
    dst_idx = edge_index[1]
    seg = _segment_sums(edge_attributes, src_idx, dst_idx)
    W_rec = W[:D_EDGE]
    W_sen = W[D_EDGE : 2 * D_EDGE]
    W_node = W[2 * D_EDGE : 2 * D_EDGE + D_NODE]
    W_g = W[2 * D_EDGE + D_NODE :]
    g_row = global_attributes[None, :]
    b_row = b[None, :]
    return _node_update(seg, node_attributes, g_row, W_rec, W_sen, W_node, W_g, b_row)


# speedup vs baseline: 9.8979x; 1.0000x over previous
import functools

import jax
import jax.numpy as jnp
from jax import lax
from jax.experimental import pallas as pl
from jax.experimental.pallas import tpu as pltpu
from jax.experimental.pallas import tpu_sc as plsc

N_NODES = 100000
N_EDGES = 3200000
D_NODE = 128
D_EDGE = 16
D_GLOBAL = 32

NC = 2
NS = 16
EDGES_PER_TILE = N_EDGES // NS
CHUNK = 800
N_CHUNKS = EDGES_PER_TILE // CHUNK
N_PAD = 100096
ROWS_PER_TILE = N_PAD // NS
ZROWS = 136
NZ = ROWS_PER_TILE // ZROWS


def _segment_sums(edge_attributes, src_idx, dst_idx):
    mesh = plsc.VectorSubcoreMesh(core_axis_name="c", subcore_axis_name="s")

    @functools.partial(
        pl.kernel,
        mesh=mesh,
        out_type=jax.ShapeDtypeStruct((2, N_PAD, D_EDGE), jnp.float32),
        scratch_types=[
            pltpu.VMEM_SHARED((N_PAD, D_EDGE), jnp.float32),
            pltpu.VMEM((CHUNK, D_EDGE), jnp.float32),
            pltpu.VMEM((CHUNK, D_EDGE), jnp.float32),
            pltpu.VMEM((CHUNK,), jnp.int32),
            pltpu.VMEM((CHUNK,), jnp.int32),
            pltpu.VMEM((ZROWS, D_EDGE), jnp.float32),
            pltpu.SemaphoreType.DMA,
            pltpu.SemaphoreType.DMA,
        ],
        compiler_params=pltpu.CompilerParams(use_tc_tiling_on_sc=False),
    )
    def seg(edge_hbm, src_hbm, dst_hbm, out_hbm, acc, ebuf0, ebuf1, ibuf0,
            ibuf1, zbuf, sem0, sem1):
        c = lax.axis_index("c")
        s = lax.axis_index("s")

        def zrow(i, carry):
            zbuf[i, :] = jnp.zeros((16,), jnp.float32)
            return carry

        lax.fori_loop(0, ZROWS, zrow, 0)
        row0 = s * ROWS_PER_TILE

        def zcopy(j, carry):
            pltpu.sync_copy(zbuf, acc.at[pl.ds(row0 + j * ZROWS, ZROWS), :])
            return carry

        lax.fori_loop(0, NZ, zcopy, 0)
        plsc.subcore_barrier()

        e_base = s * EDGES_PER_TILE
        ebufs = (ebuf0, ebuf1)
        ibufs = (ibuf0, ibuf1)
        sems = (sem0, sem1)

        def run_direction(idx_hbm):
            def start_load(k, slot):
                e0 = e_base + k * CHUNK
                pltpu.async_copy(edge_hbm.at[pl.ds(e0, CHUNK), :], ebufs[slot],
                                 sems[slot])
                pltpu.async_copy(idx_hbm.at[pl.ds(e0, CHUNK)], ibufs[slot],
                                 sems[slot])

            def wait_load(slot):
                pltpu.make_async_copy(edge_hbm.at[pl.ds(0, CHUNK), :],
                                      ebufs[slot], sems[slot]).wait()
                pltpu.make_async_copy(idx_hbm.at[pl.ds(0, CHUNK)],
                                      ibufs[slot], sems[slot]).wait()

            start_load(0, 0)

            def pair_body(g, carry):
                base = g * 2
                for slot in (0, 1):
                    k = base + slot
                    wait_load(slot)
                    pltpu.sync_copy(ebufs[slot], acc.at[ibufs[slot]], add=True)
                    nxt = k + 2

                    @pl.when(nxt < N_CHUNKS)
                    def _():
                        start_load(nxt, slot)

                return carry

            start_load(1, 1)
            lax.fori_loop(0, N_CHUNKS // 2, pair_body, 0)

        pl.when(c == 0)(lambda: run_direction(dst_hbm))
        pl.when(c == 1)(lambda: run_direction(src_hbm))
        plsc.subcore_barrier()

        pltpu.sync_copy(
            acc.at[pl.ds(row0, ROWS_PER_TILE), :],
            out_hbm.at[c, pl.ds(row0, ROWS_PER_TILE), :],
        )

    return seg(edge_attributes, src_idx, dst_idx)


def _node_update(seg, node_attributes, g_row, W_rec, W_sen, W_node, W_g, b_row):
    B = 4000
    grid = (N_NODES // B,)

    def mm(rec_ref, sen_ref, node_ref, g_ref, wr_ref, ws_ref, wn_ref, wg_ref,
           b_ref, out_ref):
        const = (
            jnp.dot(g_ref[...], wg_ref[...], preferred_element_type=jnp.float32)
            + b_ref[...]
        )
        out_ref[...] = (
            jnp.dot(node_ref[...], wn_ref[...], preferred_element_type=jnp.float32)
            + jnp.dot(rec_ref[0], wr_ref[...], preferred_element_type=jnp.float32)
            + jnp.dot(sen_ref[0], ws_ref[...], preferred_element_type=jnp.float32)
            + const
        )

    return pl.pallas_call(
        mm,
        grid=grid,
        in_specs=[
            pl.BlockSpec((1, B, D_EDGE), lambda i: (0, i, 0)),
            pl.BlockSpec((1, B, D_EDGE), lambda i: (1, i, 0)),
            pl.BlockSpec((B, D_NODE), lambda i: (i, 0)),
            pl.BlockSpec((1, D_GLOBAL), lambda i: (0, 0)),
            pl.BlockSpec((D_EDGE, D_NODE), lambda i: (0, 0)),
            pl.BlockSpec((D_EDGE, D_NODE), lambda i: (0, 0)),
            pl.BlockSpec((D_NODE, D_NODE), lambda i: (0, 0)),
            pl.BlockSpec((D_GLOBAL, D_NODE), lambda i: (0, 0)),
            pl.BlockSpec((1, D_NODE), lambda i: (0, 0)),
        ],
        out_specs=pl.BlockSpec((B, D_NODE), lambda i: (i, 0)),
        out_shape=jax.ShapeDtypeStruct((N_NODES, D_NODE), jnp.float32),
    )(seg, seg, node_attributes, g_row, W_rec, W_sen, W_node, W_g, b_row)


def kernel(node_attributes, edge_attributes, global_attributes, edge_index, W, b):
    src_idx = edge_index[0]
    dst_idx = edge_index[1]
    edge_c = jnp.copy(edge_attributes)
    seg = _segment_sums(edge_c, src_idx, dst_idx)
    W_rec = W[:D_EDGE]
    W_sen = W[D_EDGE : 2 * D_EDGE]
    W_node = W[2 * D_EDGE : 2 * D_EDGE + D_NODE]
    W_g = W[2 * D_EDGE + D_NODE :]
    g_row = global_attributes[None, :]
    b_row = b[None, :]
    return _node_update(seg, node_attributes, g_row, W_rec, W_sen, W_node, W_g, b_row)


# speedup vs baseline: 16.2764x; 1.6444x over previous
import functools

import jax
import jax.numpy as jnp
from jax import lax
from jax.experimental import pallas as pl
from jax.experimental.pallas import tpu as pltpu
from jax.experimental.pallas import tpu_sc as plsc

N_NODES = 100000
N_EDGES = 3200000
D_NODE = 128
D_EDGE = 16
D_GLOBAL = 32

NC = 2
NS = 16
EDGES_PER_TILE = N_EDGES // NS
CHUNK = 800
N_CHUNKS = EDGES_PER_TILE // CHUNK
N_PAD = 100096
ROWS_PER_TILE = N_PAD // NS
ZROWS = 136
NZ = ROWS_PER_TILE // ZROWS


def _segment_sums(edge_attributes, src_idx, dst_idx):
    mesh = plsc.VectorSubcoreMesh(core_axis_name="c", subcore_axis_name="s")

    @functools.partial(
        pl.kernel,
        mesh=mesh,
        out_type=jax.ShapeDtypeStruct((2, N_PAD, D_EDGE), jnp.float32),
        scratch_types=[
            pltpu.VMEM_SHARED((N_PAD, D_EDGE), jnp.float32),
            pltpu.VMEM((CHUNK, D_EDGE), jnp.float32),
            pltpu.VMEM((CHUNK, D_EDGE), jnp.float32),
            pltpu.VMEM((CHUNK,), jnp.int32),
            pltpu.VMEM((CHUNK,), jnp.int32),
            pltpu.VMEM((ZROWS, D_EDGE), jnp.float32),
            pltpu.SemaphoreType.DMA,
            pltpu.SemaphoreType.DMA,
        ],
        compiler_params=pltpu.CompilerParams(use_tc_tiling_on_sc=False),
    )
    def seg(edge_hbm, src_hbm, dst_hbm, out_hbm, acc, ebuf0, ebuf1, ibuf0,
            ibuf1, zbuf, sem0, sem1):
        c = lax.axis_index("c")
        s = lax.axis_index("s")

        def zrow(i, carry):
            zbuf[i, :] = jnp.zeros((16,), jnp.float32)
            return carry

        lax.fori_loop(0, ZROWS, zrow, 0)
        row0 = s * ROWS_PER_TILE

        def zcopy(j, carry):
            pltpu.sync_copy(zbuf, acc.at[pl.ds(row0 + j * ZROWS, ZROWS), :])
            return carry

        lax.fori_loop(0, NZ, zcopy, 0)
        plsc.subcore_barrier()

        e_base = s * EDGES_PER_TILE
        ebufs = (ebuf0, ebuf1)
        ibufs = (ibuf0, ibuf1)
        sems = (sem0, sem1)

        def run_direction(idx_hbm):
            def start_load(k, slot):
                e0 = e_base + k * CHUNK
                pltpu.async_copy(edge_hbm.at[pl.ds(e0, CHUNK), :], ebufs[slot],
                                 sems[slot])
                pltpu.async_copy(idx_hbm.at[pl.ds(e0, CHUNK)], ibufs[slot],
                                 sems[slot])

            def wait_load(slot):
                pltpu.make_async_copy(edge_hbm.at[pl.ds(0, CHUNK), :],
                                      ebufs[slot], sems[slot]).wait()
                pltpu.make_async_copy(idx_hbm.at[pl.ds(0, CHUNK)],
                                      ibufs[slot], sems[slot]).wait()

            start_load(0, 0)

            def pair_body(g, carry):
                base = g * 2
                for slot in (0, 1):
                    k = base + slot
                    wait_load(slot)
                    pltpu.sync_copy(ebufs[slot], acc.at[ibufs[slot]], add=True)
                    nxt = k + 2

                    @pl.when(nxt < N_CHUNKS)
                    def _():
                        start_load(nxt, slot)

                return carry

            start_load(1, 1)
            lax.fori_loop(0, N_CHUNKS // 2, pair_body, 0)

        pl.when(c == 0)(lambda: run_direction(dst_hbm))
        pl.when(c == 1)(lambda: run_direction(src_hbm))
        plsc.subcore_barrier()

        pltpu.sync_copy(
            acc.at[pl.ds(row0, ROWS_PER_TILE), :],
            out_hbm.at[c, pl.ds(row0, ROWS_PER_TILE), :],
        )

    return seg(edge_attributes, src_idx, dst_idx)


N_GROUPS = N_EDGES // 128
GW_BASE = N_GROUPS // (NC * NS)
GW_EXTRA = N_GROUPS % (NC * NS)


def _edge_rows_sc(edge_t):
    mesh = plsc.VectorSubcoreMesh(core_axis_name="c", subcore_axis_name="s")

    @functools.partial(
        pl.kernel,
        mesh=mesh,
        out_type=jax.ShapeDtypeStruct((N_EDGES * D_EDGE,), jnp.float32),
        scratch_types=[
            pltpu.VMEM((D_EDGE, 128), jnp.float32),
            pltpu.VMEM((D_EDGE, 128), jnp.float32),
            pltpu.VMEM((128 * D_EDGE,), jnp.float32),
            pltpu.VMEM((128 * D_EDGE,), jnp.float32),
            pltpu.SemaphoreType.DMA,
            pltpu.SemaphoreType.DMA,
            pltpu.SemaphoreType.DMA,
            pltpu.SemaphoreType.DMA,
        ],
        compiler_params=pltpu.CompilerParams(needs_layout_passes=False),
    )
    def tr(et_hbm, out_hbm, slab0, slab1, pk0, pk1, si0, si1, so0, so1):
        c = lax.axis_index("c")
        s = lax.axis_index("s")
        w = s * NC + c
        g0 = w * GW_BASE + jnp.minimum(w, GW_EXTRA)
        trip = GW_BASE + jnp.where(w < GW_EXTRA, 1, 0)

        slabs = (slab0, slab1)
        pks = (pk0, pk1)
        sin = (si0, si1)
        sout = (so0, so1)
        ibase = lax.iota(jnp.int32, 16) * D_EDGE

        def start_load(k, slot):
            pltpu.async_copy(et_hbm.at[:, pl.ds((g0 + k) * 128, 128)],
                             slabs[slot], sin[slot])

        def wait_load(slot):
            pltpu.make_async_copy(et_hbm.at[:, pl.ds(0, 128)], slabs[slot],
                                  sin[slot]).wait()

        def pack(slot):
            slab = slabs[slot]
            pk = pks[slot]
            for j in range(D_EDGE):
                for m in range(8):
                    vals = slab[j, pl.ds(16 * m, 16)]
                    plsc.store_scatter(pk, [ibase + (256 * m + j)], vals)

        def start_store(k, slot):
            pltpu.async_copy(pks[slot], out_hbm.at[pl.ds((g0 + k) * 2048, 2048)],
                             sout[slot])

        def wait_store(slot):
            pltpu.make_async_copy(pks[slot], out_hbm.at[pl.ds(0, 2048)],
                                  sout[slot]).wait()

        start_load(0, 0)

        @pl.when(trip > 1)
        def _():
            start_load(1, 1)

        def body(k, carry):
            slot = lax.rem(k, 2)
            for sl in (0, 1):
                @pl.when(slot == sl)
                def _():
                    wait_load(sl)

                    @pl.when(k >= 2)
                    def _():
                        wait_store(sl)

                    pack(sl)
                    start_store(k, sl)

                    @pl.when(k + 2 < trip)
                    def _():
                        start_load(k + 2, sl)

            return carry

        lax.fori_loop(0, trip, body, 0)

        for sl in (0, 1):
            @pl.when(lax.rem(trip - 1, 2) == sl)
            def _():
                wait_store(sl)

            @pl.when(lax.rem(trip - 2, 2) == sl)
            def _():
                wait_store(sl)

    return tr(edge_t)


def _edge_rows(edge_t):
    edge_flat = _edge_rows_sc(edge_t)
    return edge_flat.reshape(N_EDGES // 8, 128)


def _node_update(seg, node_attributes, g_row, W_rec, W_sen, W_node, W_g, b_row):
    B = 4000
    grid = (N_NODES // B,)

    def mm(rec_ref, sen_ref, node_ref, g_ref, wr_ref, ws_ref, wn_ref, wg_ref,
           b_ref, out_ref):
        const = (
            jnp.dot(g_ref[...], wg_ref[...], preferred_element_type=jnp.float32)
            + b_ref[...]
        )
        out_ref[...] = (
            jnp.dot(node_ref[...], wn_ref[...], preferred_element_type=jnp.float32)
            + jnp.dot(rec_ref[0], wr_ref[...], preferred_element_type=jnp.float32)
            + jnp.dot(sen_ref[0], ws_ref[...], preferred_element_type=jnp.float32)
            + const
        )

    return pl.pallas_call(
        mm,
        grid=grid,
        in_specs=[
            pl.BlockSpec((1, B, D_EDGE), lambda i: (0, i, 0)),
            pl.BlockSpec((1, B, D_EDGE), lambda i: (1, i, 0)),
            pl.BlockSpec((B, D_NODE), lambda i: (i, 0)),
            pl.BlockSpec((1, D_GLOBAL), lambda i: (0, 0)),
            pl.BlockSpec((D_EDGE, D_NODE), lambda i: (0, 0)),
            pl.BlockSpec((D_EDGE, D_NODE), lambda i: (0, 0)),
            pl.BlockSpec((D_NODE, D_NODE), lambda i: (0, 0)),
            pl.BlockSpec((D_GLOBAL, D_NODE), lambda i: (0, 0)),
            pl.BlockSpec((1, D_NODE), lambda i: (0, 0)),
        ],
        out_specs=pl.BlockSpec((B, D_NODE), lambda i: (i, 0)),
        out_shape=jax.ShapeDtypeStruct((N_NODES, D_NODE), jnp.float32),
    )(seg, seg, node_attributes, g_row, W_rec, W_sen, W_node, W_g, b_row)


def kernel(node_attributes, edge_attributes, global_attributes, edge_index, W, b):
    src_idx = edge_index[0]
    dst_idx = edge_index[1]
    edge_packed = _edge_rows(edge_attributes.T)
    edge_c = edge_packed.reshape(N_EDGES, D_EDGE)
    seg = _segment_sums(edge_c, src_idx, dst_idx)
    W_rec = W[:D_EDGE]
    W_sen = W[D_EDGE : 2 * D_EDGE]
    W_node = W[2 * D_EDGE : 2 * D_EDGE + D_NODE]
    W_g = W[2 * D_EDGE + D_NODE :]
    g_row = global_attributes[None, :]
    b_row = b[None, :]
    return _node_update(seg, node_attributes, g_row, W_rec, W_sen, W_node, W_g, b_row)


# speedup vs baseline: 17.2784x; 1.0616x over previous
import functools

import jax
import jax.numpy as jnp
from jax import lax
from jax.experimental import pallas as pl
from jax.experimental.pallas import tpu as pltpu
from jax.experimental.pallas import tpu_sc as plsc

N_NODES = 100000
N_EDGES = 3200000
D_NODE = 128
D_EDGE = 16
D_GLOBAL = 32

NC = 2
NS = 16
EDGES_PER_TILE = N_EDGES // NS
CHUNK = 800
N_CHUNKS = EDGES_PER_TILE // CHUNK
N_PAD = 100096
ROWS_PER_TILE = N_PAD // NS
ZROWS = 136
NZ = ROWS_PER_TILE // ZROWS


def _segment_sums(edge_a, edge_b, src_idx, dst_idx):
    mesh = plsc.VectorSubcoreMesh(core_axis_name="c", subcore_axis_name="s")

    @functools.partial(
        pl.kernel,
        mesh=mesh,
        out_type=jax.ShapeDtypeStruct((2, N_PAD, D_EDGE), jnp.float32),
        scratch_types=[
            pltpu.VMEM_SHARED((N_PAD, D_EDGE), jnp.float32),
            pltpu.VMEM((CHUNK, D_EDGE), jnp.float32),
            pltpu.VMEM((CHUNK, D_EDGE), jnp.float32),
            pltpu.VMEM((CHUNK,), jnp.int32),
            pltpu.VMEM((CHUNK,), jnp.int32),
            pltpu.VMEM((ZROWS, D_EDGE), jnp.float32),
            pltpu.SemaphoreType.DMA,
            pltpu.SemaphoreType.DMA,
        ],
        compiler_params=pltpu.CompilerParams(use_tc_tiling_on_sc=False),
    )
    def seg(ea_hbm, eb_hbm, src_hbm, dst_hbm, out_hbm, acc, ebuf0, ebuf1,
            ibuf0, ibuf1, zbuf, sem0, sem1):
        c = lax.axis_index("c")
        s = lax.axis_index("s")

        def zrow(i, carry):
            zbuf[i, :] = jnp.zeros((16,), jnp.float32)
            return carry

        lax.fori_loop(0, ZROWS, zrow, 0)
        row0 = s * ROWS_PER_TILE

        def zcopy(j, carry):
            pltpu.sync_copy(zbuf, acc.at[pl.ds(row0 + j * ZROWS, ZROWS), :])
            return carry

        lax.fori_loop(0, NZ, zcopy, 0)
        plsc.subcore_barrier()

        e_base = s * EDGES_PER_TILE
        ebufs = (ebuf0, ebuf1)
        ibufs = (ibuf0, ibuf1)
        sems = (sem0, sem1)

        def run_direction(idx_hbm, e_hbm, e_off):
            def start_load(k, slot):
                e0 = e_base + k * CHUNK
                pltpu.async_copy(e_hbm.at[pl.ds(e0 - e_off, CHUNK), :],
                                 ebufs[slot], sems[slot])
                pltpu.async_copy(idx_hbm.at[pl.ds(e0, CHUNK)], ibufs[slot],
                                 sems[slot])

            def wait_load(slot):
                pltpu.make_async_copy(e_hbm.at[pl.ds(0, CHUNK), :],
                                      ebufs[slot], sems[slot]).wait()
                pltpu.make_async_copy(idx_hbm.at[pl.ds(0, CHUNK)],
                                      ibufs[slot], sems[slot]).wait()

            start_load(0, 0)

            def pair_body(g, carry):
                base = g * 2
                for slot in (0, 1):
                    k = base + slot
                    wait_load(slot)
                    pltpu.sync_copy(ebufs[slot], acc.at[ibufs[slot]], add=True)
                    nxt = k + 2

                    @pl.when(nxt < N_CHUNKS)
                    def _():
                        start_load(nxt, slot)

                return carry

            start_load(1, 1)
            lax.fori_loop(0, N_CHUNKS // 2, pair_body, 0)

        half = s // 8
        for cc, idxr in ((0, dst_hbm), (1, src_hbm)):
            for hh, eref, off in ((0, ea_hbm, 0), (1, eb_hbm, N_EDGES // 2)):
                pl.when((c == cc) & (half == hh))(
                    lambda idxr=idxr, eref=eref, off=off: run_direction(
                        idxr, eref, off))
        plsc.subcore_barrier()

        pltpu.sync_copy(
            acc.at[pl.ds(row0, ROWS_PER_TILE), :],
            out_hbm.at[c, pl.ds(row0, ROWS_PER_TILE), :],
        )

    return seg(edge_a, edge_b, src_idx, dst_idx)


N_HALF = N_EDGES // 2
N_GROUPS = N_HALF // 128
GW_BASE = N_GROUPS // (NC * NS)
GW_EXTRA = N_GROUPS % (NC * NS)
G_OFF = N_HALF // 128


def _edge_rows_sc(edge_t):
    mesh = plsc.VectorSubcoreMesh(core_axis_name="c", subcore_axis_name="s")

    @functools.partial(
        pl.kernel,
        mesh=mesh,
        out_type=jax.ShapeDtypeStruct((N_HALF * D_EDGE,), jnp.float32),
        scratch_types=[
            pltpu.VMEM((D_EDGE, 128), jnp.float32),
            pltpu.VMEM((D_EDGE, 128), jnp.float32),
            pltpu.VMEM((128 * D_EDGE,), jnp.float32),
            pltpu.VMEM((128 * D_EDGE,), jnp.float32),
            pltpu.SemaphoreType.DMA,
            pltpu.SemaphoreType.DMA,
            pltpu.SemaphoreType.DMA,
            pltpu.SemaphoreType.DMA,
        ],
        compiler_params=pltpu.CompilerParams(needs_layout_passes=False),
    )
    def tr(et_hbm, out_hbm, slab0, slab1, pk0, pk1, si0, si1, so0, so1):
        c = lax.axis_index("c")
        s = lax.axis_index("s")
        w = s * NC + c
        g0 = w * GW_BASE + jnp.minimum(w, GW_EXTRA)
        trip = GW_BASE + jnp.where(w < GW_EXTRA, 1, 0)

        slabs = (slab0, slab1)
        pks = (pk0, pk1)
        sin = (si0, si1)
        sout = (so0, so1)
        ibase = lax.iota(jnp.int32, 16) * D_EDGE

        def start_load(k, slot):
            pltpu.async_copy(et_hbm.at[:, pl.ds((G_OFF + g0 + k) * 128, 128)],
                             slabs[slot], sin[slot])

        def wait_load(slot):
            pltpu.make_async_copy(et_hbm.at[:, pl.ds(0, 128)], slabs[slot],
                                  sin[slot]).wait()

        def pack(slot):
            slab = slabs[slot]
            pk = pks[slot]
            for j in range(D_EDGE):
                for m in range(8):
                    vals = slab[j, pl.ds(16 * m, 16)]
                    plsc.store_scatter(pk, [ibase + (256 * m + j)], vals)

        def start_store(k, slot):
            pltpu.async_copy(pks[slot], out_hbm.at[pl.ds((g0 + k) * 2048, 2048)],
                             sout[slot])

        def wait_store(slot):
            pltpu.make_async_copy(pks[slot], out_hbm.at[pl.ds(0, 2048)],
                                  sout[slot]).wait()

        start_load(0, 0)

        @pl.when(trip > 1)
        def _():
            start_load(1, 1)

        def body(k, carry):
            slot = lax.rem(k, 2)
            for sl in (0, 1):
                @pl.when(slot == sl)
                def _():
                    wait_load(sl)

                    @pl.when(k >= 2)
                    def _():
                        wait_store(sl)

                    pack(sl)
                    start_store(k, sl)

                    @pl.when(k + 2 < trip)
                    def _():
                        start_load(k + 2, sl)

            return carry

        lax.fori_loop(0, trip, body, 0)

        for sl in (0, 1):
            @pl.when(lax.rem(trip - 1, 2) == sl)
            def _():
                wait_store(sl)

            @pl.when(lax.rem(trip - 2, 2) == sl)
            def _():
                wait_store(sl)

    return tr(edge_t)


TBK = 12800


def _edge_rows_tc(edge_t):
    def tr(x_ref, o_ref):
        x = x_ref[...]
        y = jnp.swapaxes(x, 0, 1)
        y3 = y.reshape(TBK // 8, 8, 16)
        o_ref[...] = jnp.concatenate([y3[:, n, :] for n in range(8)], axis=1)

    return pl.pallas_call(
        tr,
        grid=(N_HALF // TBK,),
        in_specs=[pl.BlockSpec((D_EDGE, TBK), lambda i: (0, i))],
        out_specs=pl.BlockSpec((TBK // 8, 128), lambda i: (i, 0)),
        out_shape=jax.ShapeDtypeStruct((N_HALF // 8, 128), jnp.float32),
    )(edge_t)


def _node_update(seg, node_attributes, g_row, W_rec, W_sen, W_node, W_g, b_row):
    B = 4000
    grid = (N_NODES // B,)

    def mm(rec_ref, sen_ref, node_ref, g_ref, wr_ref, ws_ref, wn_ref, wg_ref,
           b_ref, out_ref):
        const = (
            jnp.dot(g_ref[...], wg_ref[...], preferred_element_type=jnp.float32)
            + b_ref[...]
        )
        out_ref[...] = (
            jnp.dot(node_ref[...], wn_ref[...], preferred_element_type=jnp.float32)
            + jnp.dot(rec_ref[0], wr_ref[...], preferred_element_type=jnp.float32)
            + jnp.dot(sen_ref[0], ws_ref[...], preferred_element_type=jnp.float32)
            + const
        )

    return pl.pallas_call(
        mm,
        grid=grid,
        in_specs=[
            pl.BlockSpec((1, B, D_EDGE), lambda i: (0, i, 0)),
            pl.BlockSpec((1, B, D_EDGE), lambda i: (1, i, 0)),
            pl.BlockSpec((B, D_NODE), lambda i: (i, 0)),
            pl.BlockSpec((1, D_GLOBAL), lambda i: (0, 0)),
            pl.BlockSpec((D_EDGE, D_NODE), lambda i: (0, 0)),
            pl.BlockSpec((D_EDGE, D_NODE), lambda i: (0, 0)),
            pl.BlockSpec((D_NODE, D_NODE), lambda i: (0, 0)),
            pl.BlockSpec((D_GLOBAL, D_NODE), lambda i: (0, 0)),
            pl.BlockSpec((1, D_NODE), lambda i: (0, 0)),
        ],
        out_specs=pl.BlockSpec((B, D_NODE), lambda i: (i, 0)),
        out_shape=jax.ShapeDtypeStruct((N_NODES, D_NODE), jnp.float32),
    )(seg, seg, node_attributes, g_row, W_rec, W_sen, W_node, W_g, b_row)


def kernel(node_attributes, edge_attributes, global_attributes, edge_index, W, b):
    src_idx = edge_index[0]
    dst_idx = edge_index[1]
    et = edge_attributes.T
    edge_a = _edge_rows_tc(et).reshape(N_HALF, D_EDGE)
    edge_b = _edge_rows_sc(et).reshape(N_HALF, D_EDGE)
    seg = _segment_sums(edge_a, edge_b, src_idx, dst_idx)
    W_rec = W[:D_EDGE]
    W_sen = W[D_EDGE : 2 * D_EDGE]
    W_node = W[2 * D_EDGE : 2 * D_EDGE + D_NODE]
    W_g = W[2 * D_EDGE + D_NODE :]
    g_row = global_attributes[None, :]
    b_row = b[None, :]
    return _node_update(seg, node_attributes, g_row, W_rec, W_sen, W_node, W_g, b_row)


# speedup vs baseline: 19.8979x; 1.1516x over previous
import functools

import jax
import jax.numpy as jnp
from jax import lax
from jax.experimental import pallas as pl
from jax.experimental.pallas import tpu as pltpu
from jax.experimental.pallas import tpu_sc as plsc

N_NODES = 100000
N_EDGES = 3200000
D_NODE = 128
D_EDGE = 16
D_GLOBAL = 32

NC = 2
NS = 16
EDGES_PER_TILE = N_EDGES // NS
CHUNK = 800
N_CHUNKS = EDGES_PER_TILE // CHUNK
N_PAD = 100096
ROWS_PER_TILE = N_PAD // NS
ZROWS = 136
NZ = ROWS_PER_TILE // ZROWS


def _segment_sums(edge_a, edge_b, src_idx, dst_idx):
    mesh = plsc.VectorSubcoreMesh(core_axis_name="c", subcore_axis_name="s")

    @functools.partial(
        pl.kernel,
        mesh=mesh,
        out_type=jax.ShapeDtypeStruct((2, N_PAD, D_EDGE), jnp.float32),
        scratch_types=[
            pltpu.VMEM_SHARED((N_PAD, D_EDGE), jnp.float32),
            pltpu.VMEM((CHUNK, D_EDGE), jnp.float32),
            pltpu.VMEM((CHUNK, D_EDGE), jnp.float32),
            pltpu.VMEM((CHUNK,), jnp.int32),
            pltpu.VMEM((CHUNK,), jnp.int32),
            pltpu.VMEM((ZROWS, D_EDGE), jnp.float32),
            pltpu.SemaphoreType.DMA,
            pltpu.SemaphoreType.DMA,
        ],
        compiler_params=pltpu.CompilerParams(use_tc_tiling_on_sc=False),
    )
    def seg(ea_hbm, eb_hbm, src_hbm, dst_hbm, out_hbm, acc, ebuf0, ebuf1,
            ibuf0, ibuf1, zbuf, sem0, sem1):
        c = lax.axis_index("c")
        s = lax.axis_index("s")

        def zrow(i, carry):
            zbuf[i, :] = jnp.zeros((16,), jnp.float32)
            return carry

        lax.fori_loop(0, ZROWS, zrow, 0)
        row0 = s * ROWS_PER_TILE

        def zcopy(j, carry):
            pltpu.sync_copy(zbuf, acc.at[pl.ds(row0 + j * ZROWS, ZROWS), :])
            return carry

        lax.fori_loop(0, NZ, zcopy, 0)
        plsc.subcore_barrier()

        e_base = s * EDGES_PER_TILE
        ebufs = (ebuf0, ebuf1)
        ibufs = (ibuf0, ibuf1)
        sems = (sem0, sem1)

        def run_direction(idx_hbm, e_hbm, e_off):
            def start_load(k, slot):
                e0 = e_base + k * CHUNK
                pltpu.async_copy(e_hbm.at[pl.ds(e0 - e_off, CHUNK), :],
                                 ebufs[slot], sems[slot])
                pltpu.async_copy(idx_hbm.at[pl.ds(e0, CHUNK)], ibufs[slot],
                                 sems[slot])

            def wait_load(slot):
                pltpu.make_async_copy(e_hbm.at[pl.ds(0, CHUNK), :],
                                      ebufs[slot], sems[slot]).wait()
                pltpu.make_async_copy(idx_hbm.at[pl.ds(0, CHUNK)],
                                      ibufs[slot], sems[slot]).wait()

            start_load(0, 0)

            def pair_body(g, carry):
                base = g * 2
                for slot in (0, 1):
                    k = base + slot
                    wait_load(slot)
                    pltpu.sync_copy(ebufs[slot], acc.at[ibufs[slot]], add=True)
                    nxt = k + 2

                    @pl.when(nxt < N_CHUNKS)
                    def _():
                        start_load(nxt, slot)

                return carry

            start_load(1, 1)
            lax.fori_loop(0, N_CHUNKS // 2, pair_body, 0)

        half = jnp.where(s < TILES_A, 0, 1)
        for cc, idxr in ((0, dst_hbm), (1, src_hbm)):
            for hh, eref, off in ((0, ea_hbm, 0), (1, eb_hbm, N_A)):
                pl.when((c == cc) & (half == hh))(
                    lambda idxr=idxr, eref=eref, off=off: run_direction(
                        idxr, eref, off))
        plsc.subcore_barrier()

        pltpu.sync_copy(
            acc.at[pl.ds(row0, ROWS_PER_TILE), :],
            out_hbm.at[c, pl.ds(row0, ROWS_PER_TILE), :],
        )

    return seg(edge_a, edge_b, src_idx, dst_idx)


N_A = 1200000
N_B = N_EDGES - N_A
N_GROUPS = N_B // 128
GW_BASE = N_GROUPS // (NC * NS)
GW_EXTRA = N_GROUPS % (NC * NS)
G_OFF = N_A // 128
TILES_A = N_A // EDGES_PER_TILE


def _edge_rows_sc(edge_t):
    mesh = plsc.VectorSubcoreMesh(core_axis_name="c", subcore_axis_name="s")

    @functools.partial(
        pl.kernel,
        mesh=mesh,
        out_type=jax.ShapeDtypeStruct((N_B * D_EDGE,), jnp.float32),
        scratch_types=[
            pltpu.VMEM((D_EDGE, 128), jnp.float32),
            pltpu.VMEM((D_EDGE, 128), jnp.float32),
            pltpu.VMEM((128 * D_EDGE,), jnp.float32),
            pltpu.VMEM((128 * D_EDGE,), jnp.float32),
            pltpu.SemaphoreType.DMA,
            pltpu.SemaphoreType.DMA,
            pltpu.SemaphoreType.DMA,
            pltpu.SemaphoreType.DMA,
        ],
        compiler_params=pltpu.CompilerParams(needs_layout_passes=False),
    )
    def tr(et_hbm, out_hbm, slab0, slab1, pk0, pk1, si0, si1, so0, so1):
        c = lax.axis_index("c")
        s = lax.axis_index("s")
        w = s * NC + c
        g0 = w * GW_BASE + jnp.minimum(w, GW_EXTRA)
        trip = GW_BASE + jnp.where(w < GW_EXTRA, 1, 0)

        slabs = (slab0, slab1)
        pks = (pk0, pk1)
        sin = (si0, si1)
        sout = (so0, so1)
        ibase = lax.iota(jnp.int32, 16) * D_EDGE

        def start_load(k, slot):
            pltpu.async_copy(et_hbm.at[:, pl.ds((G_OFF + g0 + k) * 128, 128)],
                             slabs[slot], sin[slot])

        def wait_load(slot):
            pltpu.make_async_copy(et_hbm.at[:, pl.ds(0, 128)], slabs[slot],
                                  sin[slot]).wait()

        def pack(slot):
            slab = slabs[slot]
            pk = pks[slot]
            for j in range(D_EDGE):
                for m in range(8):
                    vals = slab[j, pl.ds(16 * m, 16)]
                    plsc.store_scatter(pk, [ibase + (256 * m + j)], vals)

        def start_store(k, slot):
            pltpu.async_copy(pks[slot], out_hbm.at[pl.ds((g0 + k) * 2048, 2048)],
                             sout[slot])

        def wait_store(slot):
            pltpu.make_async_copy(pks[slot], out_hbm.at[pl.ds(0, 2048)],
                                  sout[slot]).wait()

        start_load(0, 0)

        @pl.when(trip > 1)
        def _():
            start_load(1, 1)

        def body(k, carry):
            slot = lax.rem(k, 2)
            for sl in (0, 1):
                @pl.when(slot == sl)
                def _():
                    wait_load(sl)

                    @pl.when(k >= 2)
                    def _():
                        wait_store(sl)

                    pack(sl)
                    start_store(k, sl)

                    @pl.when(k + 2 < trip)
                    def _():
                        start_load(k + 2, sl)

            return carry

        lax.fori_loop(0, trip, body, 0)

        for sl in (0, 1):
            @pl.when(lax.rem(trip - 1, 2) == sl)
            def _():
                wait_store(sl)

            @pl.when(lax.rem(trip - 2, 2) == sl)
            def _():
                wait_store(sl)

    return tr(edge_t)


TBK = 9600


def _edge_rows_tc(edge_t):
    def tr(x_ref, o_ref):
        x = x_ref[...]
        y = jnp.swapaxes(x, 0, 1)
        y3 = y.reshape(TBK // 8, 8, 16)
        o_ref[...] = jnp.concatenate([y3[:, n, :] for n in range(8)], axis=1)

    return pl.pallas_call(
        tr,
        grid=(N_A // TBK,),
        in_specs=[pl.BlockSpec((D_EDGE, TBK), lambda i: (0, i))],
        out_specs=pl.BlockSpec((TBK // 8, 128), lambda i: (i, 0)),
        out_shape=jax.ShapeDtypeStruct((N_A // 8, 128), jnp.float32),
    )(edge_t)


def _node_update(seg, node_attributes, g_row, W_rec, W_sen, W_node, W_g, b_row):
    B = 4000
    grid = (N_NODES // B,)

    def mm(rec_ref, sen_ref, node_ref, g_ref, wr_ref, ws_ref, wn_ref, wg_ref,
           b_ref, out_ref):
        const = (
            jnp.dot(g_ref[...], wg_ref[...], preferred_element_type=jnp.float32)
            + b_ref[...]
        )
        out_ref[...] = (
            jnp.dot(node_ref[...], wn_ref[...], preferred_element_type=jnp.float32)
            + jnp.dot(rec_ref[0], wr_ref[...], preferred_element_type=jnp.float32)
            + jnp.dot(sen_ref[0], ws_ref[...], preferred_element_type=jnp.float32)
            + const
        )

    return pl.pallas_call(
        mm,
        grid=grid,
        in_specs=[
            pl.BlockSpec((1, B, D_EDGE), lambda i: (0, i, 0)),
            pl.BlockSpec((1, B, D_EDGE), lambda i: (1, i, 0)),
            pl.BlockSpec((B, D_NODE), lambda i: (i, 0)),
            pl.BlockSpec((1, D_GLOBAL), lambda i: (0, 0)),
            pl.BlockSpec((D_EDGE, D_NODE), lambda i: (0, 0)),
            pl.BlockSpec((D_EDGE, D_NODE), lambda i: (0, 0)),
            pl.BlockSpec((D_NODE, D_NODE), lambda i: (0, 0)),
            pl.BlockSpec((D_GLOBAL, D_NODE), lambda i: (0, 0)),
            pl.BlockSpec((1, D_NODE), lambda i: (0, 0)),
        ],
        out_specs=pl.BlockSpec((B, D_NODE), lambda i: (i, 0)),
        out_shape=jax.ShapeDtypeStruct((N_NODES, D_NODE), jnp.float32),
    )(seg, seg, node_attributes, g_row, W_rec, W_sen, W_node, W_g, b_row)


def kernel(node_attributes, edge_attributes, global_attributes, edge_index, W, b):
    src_idx = edge_index[0]
    dst_idx = edge_index[1]
    et = edge_attributes.T
    edge_a = _edge_rows_tc(et).reshape(N_A, D_EDGE)
    edge_b = _edge_rows_sc(et).reshape(N_B, D_EDGE)
    seg = _segment_sums(edge_a, edge_b, src_idx, dst_idx)
    W_rec = W[:D_EDGE]
    W_sen = W[D_EDGE : 2 * D_EDGE]
    W_node = W[2 * D_EDGE : 2 * D_EDGE + D_NODE]
    W_g = W[2 * D_EDGE + D_NODE :]
    g_row = global_attributes[None, :]
    b_row = b[None, :]
    return _node_update(seg, node_attributes, g_row, W_rec, W_sen, W_node, W_g, b_row)
